# baseline (speedup 1.0000x reference)
import jax
import jax.numpy as jnp
from jax.experimental import pallas as pl


def _body(m_ref, k_ref, ee_ref, out_ref):
    m = m_ref[...].astype(jnp.bfloat16)
    y = jnp.dot(m, k_ref[...], preferred_element_type=jnp.float32)
    out_ref[...] = y + ee_ref[0, :][None, :]


def tc_kernel(batch_mask, mask_emb, bm=512):
    M, N = batch_mask.shape
    _, D = mask_emb.shape
    W = N * D
    diff = (mask_emb[1] - mask_emb[0]).astype(jnp.bfloat16)
    K = (jnp.eye(N, dtype=jnp.bfloat16)[:, :, None] * diff[None, None, :]).reshape(N, W)
    ee = jnp.tile(mask_emb[0], N)[None, :]

    out = pl.pallas_call(
        _body,
        grid=(M // bm,),
        in_specs=[
            pl.BlockSpec((bm, N), lambda i: (i, 0)),
            pl.BlockSpec((N, W), lambda i: (0, 0)),
            pl.BlockSpec((1, W), lambda i: (0, 0)),
        ],
        out_specs=pl.BlockSpec((bm, W), lambda i: (i, 0)),
        out_shape=jax.ShapeDtypeStruct((M, W), jnp.float32),
    )(batch_mask, K, ee)
    return out.reshape(M, N, D)


def kernel(batch_mask, mask_emb):
    return tc_kernel(batch_mask, mask_emb)


# speedup vs baseline: 1.0573x; 1.0573x over previous
import jax
import jax.numpy as jnp
from jax.experimental import pallas as pl
from jax.experimental.pallas import tpu as pltpu


def _body(m_ref, dbig_ref, ee_ref, out_ref, k_ref):
    n, w = k_ref.shape

    @pl.when(pl.program_id(0) == 0)
    def _build_k():
        jj = jax.lax.broadcasted_iota(jnp.int32, (n, w), 0)
        ll = jax.lax.broadcasted_iota(jnp.int32, (n, w), 1)
        d = w // n
        sel = jnp.where(ll // d == jj, dbig_ref[0, :][None, :], 0.0)
        k_ref[...] = sel.astype(jnp.bfloat16)

    m = m_ref[...].astype(jnp.bfloat16)
    y = jnp.dot(m, k_ref[...], preferred_element_type=jnp.float32)
    out_ref[...] = y + ee_ref[0, :][None, :]


def kernel(batch_mask, mask_emb):
    M, N = batch_mask.shape
    _, D = mask_emb.shape
    W = N * D
    diff = mask_emb[1] - mask_emb[0]
    dbig = jnp.tile(diff, N)[None, :]
    ee = jnp.tile(mask_emb[0], N)[None, :]
    bm = 256

    out = pl.pallas_call(
        _body,
        grid=(M // bm,),
        in_specs=[
            pl.BlockSpec((bm, N), lambda i: (i, 0)),
            pl.BlockSpec((1, W), lambda i: (0, 0)),
            pl.BlockSpec((1, W), lambda i: (0, 0)),
        ],
        out_specs=pl.BlockSpec((bm, W), lambda i: (i, 0)),
        out_shape=jax.ShapeDtypeStruct((M, W), jnp.float32),
        scratch_shapes=[pltpu.VMEM((N, W), jnp.bfloat16)],
    )(batch_mask, dbig, ee)
    return out.reshape(M, N, D)
